# baseline (speedup 1.0000x reference)
import functools

import jax
import jax.numpy as jnp
from jax import lax
from jax.experimental import pallas as pl
from jax.experimental.pallas import tpu as pltpu
from jax.experimental.pallas import tpu_sc as plsc

BATCH = 4
SEQ = 2048
HIDDEN = 1024
L = 16
NW = 32
CHUNK = SEQ // (NW // BATCH)
G = 16
N_SUB = CHUNK // G


def _body(ids_hbm, mask_hbm, token_hbm, pos_hbm, out_hbm,
          ids_v, mask_v, pos_v, row0, mf_s,
          tok0, tok1, tok2, pb0, pb1,
          gs0, gs1, gs2, ps0, ps1, os0, os1, os2):
    tok_bufs = (tok0, tok1, tok2)
    pos_bufs = (pb0, pb1)
    gsem = (gs0, gs1, gs2)
    psem = (ps0, ps1)
    osem = (os0, os1, os2)

    cid = lax.axis_index("c")
    sid = lax.axis_index("s")
    wid = sid * 2 + cid
    b = wid // 8
    c = wid % 8
    s0 = c * CHUNK

    pltpu.sync_copy(ids_hbm.at[pl.ds(b * SEQ + s0, CHUNK)], ids_v)
    pltpu.sync_copy(mask_hbm.at[pl.ds(b * SEQ, SEQ)], mask_v)
    pltpu.sync_copy(pos_hbm.at[pl.ds(0, 1)], row0)

    def pstep(i, acc):
        return acc + mask_v[pl.ds(i * L, L)]
    acc = lax.fori_loop(0, s0 // L, pstep, jnp.zeros((L,), jnp.int32))
    prefix = jnp.sum(acc)

    def cstep(i, carry):
        m = mask_v[pl.ds(s0 + i * L, L)]
        cs = plsc.cumsum(m)
        dummy = SEQ + s0 + i * L + lax.iota(jnp.int32, L)
        pos = jnp.where(m == 0, dummy, carry + cs - 1)
        pos_v[pl.ds(i * L, L)] = pos
        mfv = m.astype(jnp.float32)
        for l in range(L):
            mf_s[i * L + l] = mfv[l]
        return carry + jnp.sum(m)
    lax.fori_loop(0, CHUNK // L, cstep, prefix)

    def fire_gathers(g):
        t = pltpu.async_copy(
            token_hbm.at[ids_v.at[pl.ds(g * G, G)]],
            tok_bufs[g % 3], gsem[g % 3])
        p = pltpu.async_copy(
            pos_hbm.at[pos_v.at[pl.ds(g * G, G)]],
            pos_bufs[g % 2], psem[g % 2])
        return t, p

    def add_block(g, tb, pb):
        def row_step(j, _):
            mf = mf_s[g * G + j]
            omf = 1.0 - mf
            for k in range(HIDDEN // L):
                tb[j, pl.ds(k * L, L)] = (
                    tb[j, pl.ds(k * L, L)]
                    + pb[j, pl.ds(k * L, L)] * mf
                    + row0[0, pl.ds(k * L, L)] * omf)
            return 0
        lax.fori_loop(0, G, row_step, 0)

    inflight = {0: fire_gathers(0), 1: fire_gathers(1)}
    stores = {}
    for g in range(N_SUB):
        tcp, pcp = inflight.pop(g)
        tcp.wait()
        pcp.wait()
        add_block(g, tok_bufs[g % 3], pos_bufs[g % 2])
        stores[g] = pltpu.async_copy(
            tok_bufs[g % 3],
            out_hbm.at[pl.ds(s0 + g * G, G), pl.ds(b * HIDDEN, HIDDEN)],
            osem[g % 3])
        if g + 2 < N_SUB:
            if g - 1 in stores:
                stores.pop(g - 1).wait()
            inflight[g + 2] = fire_gathers(g + 2)
    for g in sorted(stores):
        stores.pop(g).wait()


@jax.jit
def _embed(ids_flat, mask_flat, token_table, pos_table):
    mesh = plsc.VectorSubcoreMesh(core_axis_name="c", subcore_axis_name="s")
    k = functools.partial(
        pl.kernel,
        mesh=mesh,
        compiler_params=pltpu.CompilerParams(needs_layout_passes=False),
        out_type=jax.ShapeDtypeStruct((SEQ, BATCH * HIDDEN), jnp.float32),
        scratch_types=[
            pltpu.VMEM((CHUNK,), jnp.int32),
            pltpu.VMEM((SEQ,), jnp.int32),
            pltpu.VMEM((CHUNK,), jnp.int32),
            pltpu.VMEM((1, HIDDEN), jnp.float32),
            pltpu.SMEM((CHUNK,), jnp.float32),
            pltpu.VMEM((G, HIDDEN), jnp.float32),
            pltpu.VMEM((G, HIDDEN), jnp.float32),
            pltpu.VMEM((G, HIDDEN), jnp.float32),
            pltpu.VMEM((G, HIDDEN), jnp.float32),
            pltpu.VMEM((G, HIDDEN), jnp.float32),
            pltpu.SemaphoreType.DMA,
            pltpu.SemaphoreType.DMA,
            pltpu.SemaphoreType.DMA,
            pltpu.SemaphoreType.DMA,
            pltpu.SemaphoreType.DMA,
            pltpu.SemaphoreType.DMA,
            pltpu.SemaphoreType.DMA,
            pltpu.SemaphoreType.DMA,
        ],
    )(_body)
    return k(ids_flat, mask_flat, token_table, pos_table)


def kernel(input_ids, input_mask, token_table, pos_table):
    ids_flat = input_ids.reshape(-1)
    mask_flat = input_mask.astype(jnp.int32).reshape(-1)
    out = _embed(ids_flat, mask_flat, token_table, pos_table)
    return out.reshape(SEQ, BATCH, HIDDEN)


# speedup vs baseline: 1.5308x; 1.5308x over previous
import functools

import jax
import jax.numpy as jnp
from jax import lax
from jax.experimental import pallas as pl
from jax.experimental.pallas import tpu as pltpu
from jax.experimental.pallas import tpu_sc as plsc

BATCH = 4
SEQ = 2048
HIDDEN = 1024
L = 16
NW = 32
CHUNK = SEQ // (NW // BATCH)
G = 16
N_SUB = CHUNK // G


def _bcast_lane(v, j):
    idx = jnp.broadcast_to(jnp.int32(j), (L, 1))
    dnums = lax.GatherDimensionNumbers(
        offset_dims=(), collapsed_slice_dims=(0,), start_index_map=(0,))
    return lax.gather(v, idx, dnums, (1,),
                      mode=lax.GatherScatterMode.PROMISE_IN_BOUNDS)


def _body(ids_hbm, mask_hbm, token_hbm, pos_hbm, out_hbm,
          ids_v, mask_v, pos_v, row0,
          tok0, tok1, tok2, pb0, pb1,
          gs0, gs1, gs2, ps0, ps1, os0, os1, os2):
    tok_bufs = (tok0, tok1, tok2)
    pos_bufs = (pb0, pb1)
    gsem = (gs0, gs1, gs2)
    psem = (ps0, ps1)
    osem = (os0, os1, os2)

    cid = lax.axis_index("c")
    sid = lax.axis_index("s")
    wid = sid * 2 + cid
    b = wid // 8
    c = wid % 8
    s0 = c * CHUNK

    pltpu.sync_copy(ids_hbm.at[pl.ds(b * SEQ + s0, CHUNK)], ids_v)
    pltpu.sync_copy(mask_hbm.at[pl.ds(b * SEQ, SEQ)], mask_v)
    pltpu.sync_copy(pos_hbm.at[pl.ds(0, 1)], row0)

    def pstep(i, acc):
        return acc + mask_v[pl.ds(i * L, L)]
    acc = lax.fori_loop(0, s0 // L, pstep, jnp.zeros((L,), jnp.int32))
    prefix = jnp.sum(acc)

    def cstep(i, carry):
        m = mask_v[pl.ds(s0 + i * L, L)]
        cs = plsc.cumsum(m)
        dummy = SEQ + s0 + i * L + lax.iota(jnp.int32, L)
        pos = jnp.where(m == 0, dummy, carry + cs - 1)
        pos_v[pl.ds(i * L, L)] = pos
        return carry + jnp.sum(m)
    lax.fori_loop(0, CHUNK // L, cstep, prefix)

    def fire_gathers(g):
        t = pltpu.async_copy(
            token_hbm.at[ids_v.at[pl.ds(g * G, G)]],
            tok_bufs[g % 3], gsem[g % 3])
        p = pltpu.async_copy(
            pos_hbm.at[pos_v.at[pl.ds(g * G, G)]],
            pos_bufs[g % 2], psem[g % 2])
        return t, p

    def add_block(g, tb, pb):
        mfv = (mask_v[pl.ds(s0 + g * G, G)]).astype(jnp.float32)

        def col_step(k, _):
            off = pl.multiple_of(k * L, L)
            r0 = row0[0, pl.ds(off, L)]
            for j in range(G):
                mf = _bcast_lane(mfv, j)
                tb[j, pl.ds(off, L)] = (
                    tb[j, pl.ds(off, L)]
                    + pb[j, pl.ds(off, L)] * mf
                    + r0 * (1.0 - mf))
            return 0
        lax.fori_loop(0, HIDDEN // L, col_step, 0)

    inflight = {0: fire_gathers(0), 1: fire_gathers(1)}
    stores = {}
    for g in range(N_SUB):
        tcp, pcp = inflight.pop(g)
        tcp.wait()
        pcp.wait()
        add_block(g, tok_bufs[g % 3], pos_bufs[g % 2])
        stores[g] = pltpu.async_copy(
            tok_bufs[g % 3],
            out_hbm.at[pl.ds(s0 + g * G, G), pl.ds(b * HIDDEN, HIDDEN)],
            osem[g % 3])
        if g + 2 < N_SUB:
            if g - 1 in stores:
                stores.pop(g - 1).wait()
            inflight[g + 2] = fire_gathers(g + 2)
    for g in sorted(stores):
        stores.pop(g).wait()


@jax.jit
def _embed(ids_flat, mask_flat, token_table, pos_table):
    mesh = plsc.VectorSubcoreMesh(core_axis_name="c", subcore_axis_name="s")
    k = functools.partial(
        pl.kernel,
        mesh=mesh,
        compiler_params=pltpu.CompilerParams(needs_layout_passes=False),
        out_type=jax.ShapeDtypeStruct((SEQ, BATCH * HIDDEN), jnp.float32),
        scratch_types=[
            pltpu.VMEM((CHUNK,), jnp.int32),
            pltpu.VMEM((SEQ,), jnp.int32),
            pltpu.VMEM((CHUNK,), jnp.int32),
            pltpu.VMEM((1, HIDDEN), jnp.float32),
            pltpu.VMEM((G, HIDDEN), jnp.float32),
            pltpu.VMEM((G, HIDDEN), jnp.float32),
            pltpu.VMEM((G, HIDDEN), jnp.float32),
            pltpu.VMEM((G, HIDDEN), jnp.float32),
            pltpu.VMEM((G, HIDDEN), jnp.float32),
            pltpu.SemaphoreType.DMA,
            pltpu.SemaphoreType.DMA,
            pltpu.SemaphoreType.DMA,
            pltpu.SemaphoreType.DMA,
            pltpu.SemaphoreType.DMA,
            pltpu.SemaphoreType.DMA,
            pltpu.SemaphoreType.DMA,
            pltpu.SemaphoreType.DMA,
        ],
    )(_body)
    return k(ids_flat, mask_flat, token_table, pos_table)


def kernel(input_ids, input_mask, token_table, pos_table):
    ids_flat = input_ids.reshape(-1)
    mask_flat = input_mask.astype(jnp.int32).reshape(-1)
    out = _embed(ids_flat, mask_flat, token_table, pos_table)
    return out.reshape(SEQ, BATCH, HIDDEN)


# speedup vs baseline: 2.3420x; 1.5299x over previous
import functools

import jax
import jax.numpy as jnp
from jax import lax
from jax.experimental import pallas as pl
from jax.experimental.pallas import tpu as pltpu
from jax.experimental.pallas import tpu_sc as plsc

BATCH = 4
SEQ = 2048
HIDDEN = 1024
L = 16
NW = 32
CHUNK = SEQ // (NW // BATCH)
G = 16
N_SUB = CHUNK // G


def _bcast_lane(v, j):
    idx = jnp.broadcast_to(jnp.int32(j), (L, 1))
    dnums = lax.GatherDimensionNumbers(
        offset_dims=(), collapsed_slice_dims=(0,), start_index_map=(0,))
    return lax.gather(v, idx, dnums, (1,),
                      mode=lax.GatherScatterMode.PROMISE_IN_BOUNDS)


def _body(ids_hbm, mask_hbm, token_hbm, pos_hbm, out_hbm,
          ids_v, mask_v, pos_v, row0,
          tok0, tok1, tok2, pb0, pb1,
          gs0, gs1, gs2, ps0, ps1, os0, os1, os2):
    tok_bufs = (tok0, tok1, tok2)
    pos_bufs = (pb0, pb1)
    gsem = (gs0, gs1, gs2)
    psem = (ps0, ps1)
    osem = (os0, os1, os2)

    cid = lax.axis_index("c")
    sid = lax.axis_index("s")
    wid = sid * 2 + cid
    b = wid // 8
    c = wid % 8
    s0 = c * CHUNK

    pltpu.sync_copy(ids_hbm.at[pl.ds(b * SEQ + s0, CHUNK)], ids_v)
    pltpu.sync_copy(mask_hbm.at[pl.ds(b * SEQ, SEQ)], mask_v)
    pltpu.sync_copy(pos_hbm.at[pl.ds(0, 1)], row0)

    def pstep(i, acc):
        return acc + mask_v[pl.ds(i * L, L)]
    acc = lax.fori_loop(0, s0 // L, pstep, jnp.zeros((L,), jnp.int32))
    prefix = jnp.sum(acc)

    def cstep(i, carry):
        m = mask_v[pl.ds(s0 + i * L, L)]
        cs = plsc.cumsum(m)
        dummy = SEQ + s0 + i * L + lax.iota(jnp.int32, L)
        pos = jnp.where(m == 0, dummy, carry + cs - 1)
        pos_v[pl.ds(i * L, L)] = pos
        return carry + jnp.sum(m)
    lax.fori_loop(0, CHUNK // L, cstep, prefix)

    def fire_gathers(g):
        t = pltpu.async_copy(
            token_hbm.at[ids_v.at[pl.ds(g * G, G)]],
            tok_bufs[g % 3], gsem[g % 3])
        p = pltpu.async_copy(
            pos_hbm.at[pos_v.at[pl.ds(g * G, G)]],
            pos_bufs[g % 2], psem[g % 2])
        return t, p

    def add_block(g, tb, pb):
        mfv = (mask_v[pl.ds(s0 + g * G, G)]).astype(jnp.float32)

        def col_step(k, _):
            off = pl.multiple_of(k * L, L)
            r0 = row0[0, pl.ds(off, L)]
            for j in range(G):
                mf = _bcast_lane(mfv, j)
                tb[j, pl.ds(off, L)] = (
                    tb[j, pl.ds(off, L)]
                    + pb[j, pl.ds(off, L)] * mf
                    + r0 * (1.0 - mf))
            return 0
        lax.fori_loop(0, HIDDEN // L, col_step, 0)

    inflight = {0: fire_gathers(0), 1: fire_gathers(1)}
    stores = {}
    for g in range(N_SUB):
        tcp, pcp = inflight.pop(g)
        tcp.wait()
        pcp.wait()
        add_block(g, tok_bufs[g % 3], pos_bufs[g % 2])
        stores[g] = pltpu.async_copy(
            tok_bufs[g % 3],
            out_hbm.at[pl.ds(s0 + g * G, G), b],
            osem[g % 3])
        if g + 2 < N_SUB:
            if g - 1 in stores:
                stores.pop(g - 1).wait()
            inflight[g + 2] = fire_gathers(g + 2)
    for g in sorted(stores):
        stores.pop(g).wait()


@jax.jit
def _embed(ids_flat, mask_flat, token_table, pos_table):
    mesh = plsc.VectorSubcoreMesh(core_axis_name="c", subcore_axis_name="s")
    k = functools.partial(
        pl.kernel,
        mesh=mesh,
        compiler_params=pltpu.CompilerParams(needs_layout_passes=False),
        out_type=jax.ShapeDtypeStruct((SEQ, BATCH, HIDDEN), jnp.float32),
        scratch_types=[
            pltpu.VMEM((CHUNK,), jnp.int32),
            pltpu.VMEM((SEQ,), jnp.int32),
            pltpu.VMEM((CHUNK,), jnp.int32),
            pltpu.VMEM((1, HIDDEN), jnp.float32),
            pltpu.VMEM((G, HIDDEN), jnp.float32),
            pltpu.VMEM((G, HIDDEN), jnp.float32),
            pltpu.VMEM((G, HIDDEN), jnp.float32),
            pltpu.VMEM((G, HIDDEN), jnp.float32),
            pltpu.VMEM((G, HIDDEN), jnp.float32),
            pltpu.SemaphoreType.DMA,
            pltpu.SemaphoreType.DMA,
            pltpu.SemaphoreType.DMA,
            pltpu.SemaphoreType.DMA,
            pltpu.SemaphoreType.DMA,
            pltpu.SemaphoreType.DMA,
            pltpu.SemaphoreType.DMA,
            pltpu.SemaphoreType.DMA,
        ],
    )(_body)
    return k(ids_flat, mask_flat, token_table, pos_table)


def kernel(input_ids, input_mask, token_table, pos_table):
    ids_flat = input_ids.reshape(-1)
    mask_flat = input_mask.astype(jnp.int32).reshape(-1)
    return _embed(ids_flat, mask_flat, token_table, pos_table)
